# baseline (speedup 1.0000x reference)
import jax
import jax.numpy as jnp
from jax.experimental import pallas as pl
from jax.experimental.pallas import tpu as pltpu

_E = 0.57735027
_DIRS = (
    (_E, _E, _E), (_E, _E, -_E), (_E, -_E, _E), (_E, -_E, -_E),
    (-_E, _E, _E), (-_E, _E, -_E), (-_E, -_E, _E), (-_E, -_E, -_E),
)
_BIG = 3.0e38


def _loss_kernel(skel_ref, r_ref, rt_ref, nori_ref, st_ref, sn_ref, l3_ref,
                 kt_ref, nt_ref, out_ref):
    K = skel_ref[0]
    R = r_ref[0]
    RT = rt_ref[0]
    NO = nori_ref[0]
    S = st_ref[0]
    SN = sn_ref[0]
    L3 = l3_ref[0]
    KT = kt_ref[0]
    NT = nt_ref[0]

    Ns = K.shape[0]
    Np = S.shape[1]

    Sx, Sy, Sz = S[0:1], S[1:2], S[2:3]
    Kx, Ky, Kz = K[:, 0:1], K[:, 1:2], K[:, 2:3]
    Nx, Ny, Nz = NO[:, 0:1], NO[:, 1:2], NO[:, 2:3]
    KxR, KyR, KzR = KT[0:1], KT[1:2], KT[2:3]
    NxR, NyR, NzR = NT[0:1], NT[1:2], NT[2:3]

    ksq = Kx * Kx + Ky * Ky + Kz * Kz
    ksqR = KxR * KxR + KyR * KyR + KzR * KzR
    nsqR = NxR * NxR + NyR * NyR + NzR * NzR
    ssq = Sx * Sx + Sy * Sy + Sz * Sz

    d2r = (ksq + ssq) - (2.0 * Kx) * Sx
    d2r = d2r - (2.0 * Ky) * Sy
    d2r = d2r - (2.0 * Kz) * Sz
    twoR = 2.0 * R

    s1acc = jnp.zeros((Ns, 1), jnp.float32)
    cm = jnp.full((1, Np), _BIG, jnp.float32)
    for (dx, dy, dz) in _DIRS:
        kdot = Kx * dx + Ky * dy + Kz * dz
        sdot = Sx * dx + Sy * dy + Sz * dz
        a_u = twoR * kdot + R * R
        w = d2r - twoR * sdot
        s1acc = s1acc + (jnp.min(w, axis=1, keepdims=True) + a_u)
        cm = jnp.minimum(cm, jnp.min(w + a_u, axis=0, keepdims=True))
    c_s1 = jnp.sum(s1acc)
    c_s2 = jnp.sum(cm)

    d2 = jnp.maximum(d2r, 0.0)
    sd = jnp.sqrt(d2 + 1e-12)
    emat = (sd - R) ** 2
    c_p2s1 = jnp.sum(jnp.min(emat, axis=0, keepdims=True))
    c_p2s2 = jnp.sum(jnp.min(emat, axis=1, keepdims=True))

    SxC, SyC, SzC = SN[:, 0:1], SN[:, 1:2], SN[:, 2:3]
    ssqC = SxC * SxC + SyC * SyC + SzC * SzC
    d2t = (ssqC + ksqR) - (2.0 * SxC) * KxR
    d2t = d2t - (2.0 * SyC) * KyR
    d2t = d2t - (2.0 * SzC) * KzR
    d2t = jnp.maximum(d2t, 0.0)
    adots = jnp.abs(SN[:, 3:4] * NxR + SN[:, 4:5] * NyR
                    + SN[:, 5:6] * NzR)
    bits = jax.lax.bitcast_convert_type(d2t, jnp.int32)

    def bisect_body(_, carry):
        lo, hi = carry
        mid = lo + jax.lax.div(hi - lo, 2)
        cnt = jnp.sum((bits <= mid).astype(jnp.int32), axis=0,
                      keepdims=True)
        ge = cnt >= 30
        hi = jnp.where(ge, mid, hi)
        lo = jnp.where(ge, lo, mid + 1)
        return (lo, hi)

    lo0 = jnp.zeros((1, Ns), jnp.int32)
    hi0 = jnp.full((1, Ns), 0x7F800000, jnp.int32)
    _, thr = jax.lax.fori_loop(0, 31, bisect_body, (lo0, hi0))
    c_norm = jnp.sum(jnp.where(bits <= thr, adots, 0.0))

    LxC, LyC, LzC = L3[:, 0:1], L3[:, 1:2], L3[:, 2:3]
    lsqC = LxC * LxC + LyC * LyC + LzC * LzC
    q = (lsqC + ksqR) - (2.0 * LxC) * KxR
    q = q - (2.0 * LyC) * KyR
    q = q - (2.0 * LzC) * KzR
    cnl = LxC * NxR + LyC * NyR + LzC * NzR
    knR = KxR * NxR + KyR * NyR + KzR * NzR
    msum = jnp.zeros((1, Ns), jnp.float32)
    msqsum = jnp.zeros((1, Ns), jnp.float32)
    for k in range(30):
        t = k / 30.0
        a_t = (2.0 * t) * knR + (t * t) * nsqR
        m = jnp.min(q - (2.0 * t) * cnl, axis=0, keepdims=True) + a_t
        msum = msum + m
        msqsum = msqsum + m * m
    c_sw_sum = jnp.sum(msum)
    c_sw_sq = jnp.sum(msqsum)
    n_sw = jnp.float32(30 * Ns)
    sw_mean = c_sw_sum / n_sw
    c_sw_var = c_sw_sq / n_sw - sw_mean * sw_mean

    nn = jnp.sqrt(nsqR + 1e-12)
    c_ndist = jnp.sum(jnp.exp(-(nn - 0.3)) + jnp.where(nn < 0.25, 5.0, 0.0))

    d4 = ((Kx - KxR) ** 2 + (Ky - KyR) ** 2 + (Kz - KzR) ** 2)
    dnx = Nx - NxR
    dny = Ny - NyR
    dnz = Nz - NzR
    smat = jnp.sqrt(dnx * dnx + dny * dny + dnz * dnz + 1e-12)
    iota4 = jax.lax.broadcasted_iota(jnp.int32, (Ns, Ns), 1)

    wex = d4
    acc4 = jnp.zeros((Ns, 1), jnp.float32)
    for _ in range(3):
        m = jnp.min(wex, axis=1, keepdims=True)
        first = jnp.min(jnp.where(wex == m, iota4, Ns), axis=1, keepdims=True)
        sel = iota4 == first
        acc4 = acc4 + jnp.sum(jnp.where(sel, smat, 0.0), axis=1, keepdims=True)
        wex = jnp.where(sel, _BIG, wex)
    c_smooth = jnp.sum(acc4)

    c_rad = jnp.sum(RT)

    for i, v in enumerate((c_s1, c_s2, c_p2s1, c_p2s2, c_rad, c_norm,
                           c_sw_sum, c_sw_var, c_ndist, c_smooth)):
        out_ref[0:1, 0:1, i:i + 1] = jnp.reshape(v, (1, 1, 1))
    out_ref[0:1, 0:1, 10:16] = jnp.zeros((1, 1, 6), jnp.float32)


def kernel(skel_xyz, skel_radius, shape_cmb_features, skel_nori, weights,
           l3_xyz, l3_normals, shape_xyz, A, w0, w1, w2, w3, w4, w5, w6):
    B, Ns, _ = skel_xyz.shape
    Np = shape_xyz.shape[1]
    Nl = l3_xyz.shape[1]

    st = jnp.transpose(shape_xyz, (0, 2, 1))
    kt = jnp.transpose(skel_xyz, (0, 2, 1))
    nt = jnp.transpose(skel_nori, (0, 2, 1))
    rt = jnp.transpose(skel_radius, (0, 2, 1))

    out = pl.pallas_call(
        _loss_kernel,
        grid=(B,),
        in_specs=[
            pl.BlockSpec((1, Ns, 3), lambda b: (b, 0, 0)),
            pl.BlockSpec((1, Ns, 1), lambda b: (b, 0, 0)),
            pl.BlockSpec((1, 1, Ns), lambda b: (b, 0, 0)),
            pl.BlockSpec((1, Ns, 3), lambda b: (b, 0, 0)),
            pl.BlockSpec((1, 6, Np), lambda b: (b, 0, 0)),
            pl.BlockSpec((1, Np, 6), lambda b: (b, 0, 0)),
            pl.BlockSpec((1, Nl, 3), lambda b: (b, 0, 0)),
            pl.BlockSpec((1, 3, Ns), lambda b: (b, 0, 0)),
            pl.BlockSpec((1, 3, Ns), lambda b: (b, 0, 0)),
        ],
        out_specs=pl.BlockSpec((1, 1, 16), lambda b: (b, 0, 0)),
        out_shape=jax.ShapeDtypeStruct((B, 1, 16), jnp.float32),
        compiler_params=pltpu.CompilerParams(
            dimension_semantics=("parallel",)),
    )(skel_xyz, skel_radius, rt, skel_nori, st, shape_xyz, l3_xyz, kt, nt)

    o = jnp.sum(out, axis=(0, 1))
    loss_sample = o[0] / (Ns * 8.0) + o[1] / float(Np)
    loss_point2sphere = o[2] / float(Np) + o[3] / float(Ns)
    loss_radius = -o[4] / float(Ns)
    loss_normal = o[5] / 30.0 / B
    loss_skelenormal = 50.0 * o[6] / (B * Ns * 30.0) + 500.0 * o[7] / B
    loss_normaldist = o[8] / float(B * Ns)
    loss_normalsmooth = o[9] / (B * Ns * 3.0)
    return (w0 * loss_sample + loss_point2sphere * w1 + loss_radius * w2
            + loss_normal * w4 + loss_skelenormal * w5
            + w6 * loss_normaldist + 0.1 * loss_normalsmooth)


# speedup vs baseline: 1.0012x; 1.0012x over previous
import jax
import jax.numpy as jnp
from jax.experimental import pallas as pl
from jax.experimental.pallas import tpu as pltpu

_E = 0.57735027
_DIRS = (
    (_E, _E, _E), (_E, _E, -_E), (_E, -_E, _E), (_E, -_E, -_E),
    (-_E, _E, _E), (-_E, _E, -_E), (-_E, -_E, _E), (-_E, -_E, -_E),
)
_BIG = 3.0e38


def _loss_kernel(skel_ref, rt_ref, nori_ref, sn_ref, l3_ref, kt_ref, nt_ref,
                 out_ref):
    K = skel_ref[0]
    RT = rt_ref[0]
    NO = nori_ref[0]
    SN = sn_ref[0]
    L3 = l3_ref[0]
    KT = kt_ref[0]
    NT = nt_ref[0]

    Ns = K.shape[0]

    Kx, Ky, Kz = K[:, 0:1], K[:, 1:2], K[:, 2:3]
    Nx, Ny, Nz = NO[:, 0:1], NO[:, 1:2], NO[:, 2:3]
    KxR, KyR, KzR = KT[0:1], KT[1:2], KT[2:3]
    NxR, NyR, NzR = NT[0:1], NT[1:2], NT[2:3]

    ksqR = KxR * KxR + KyR * KyR + KzR * KzR
    nsqR = NxR * NxR + NyR * NyR + NzR * NzR
    twoRR = 2.0 * RT
    rsqR = RT * RT

    SxC, SyC, SzC = SN[:, 0:1], SN[:, 1:2], SN[:, 2:3]
    ssqC = SxC * SxC + SyC * SyC + SzC * SzC
    d2t = (ssqC + ksqR) - (2.0 * SxC) * KxR
    d2t = d2t - (2.0 * SyC) * KyR
    d2t = d2t - (2.0 * SzC) * KzR
    d2t = jnp.maximum(d2t, 0.0)

    s1acc = jnp.zeros((1, Ns), jnp.float32)
    for (dx, dy, dz) in _DIRS:
        kdotR = KxR * dx + KyR * dy + KzR * dz
        sdotC = SxC * dx + SyC * dy + SzC * dz
        a_u = twoRR * kdotR + rsqR
        w = d2t - sdotC * twoRR
        s1acc = s1acc + (jnp.min(w, axis=0, keepdims=True) + a_u)
    c_s1 = jnp.sum(s1acc)

    abssum = jnp.abs(SxC - KxR) + jnp.abs(SyC - KyR) + jnp.abs(SzC - KzR)
    cmmat = (d2t + rsqR) - (_E * twoRR) * abssum
    c_s2 = jnp.sum(jnp.min(cmmat, axis=1, keepdims=True))

    sdt = jnp.sqrt(d2t + 1e-12)
    emat = (sdt - RT) ** 2
    c_p2s1 = jnp.sum(jnp.min(emat, axis=1, keepdims=True))
    c_p2s2 = jnp.sum(jnp.min(emat, axis=0, keepdims=True))

    adots = jnp.abs(SN[:, 3:4] * NxR + SN[:, 4:5] * NyR
                    + SN[:, 5:6] * NzR)
    bits = jax.lax.bitcast_convert_type(d2t, jnp.int32)

    def bisect_body(_, carry):
        lo, hi = carry
        mid = lo + jax.lax.div(hi - lo, 2)
        cnt = jnp.sum((bits <= mid).astype(jnp.int32), axis=0,
                      keepdims=True)
        ge = cnt >= 30
        hi = jnp.where(ge, mid, hi)
        lo = jnp.where(ge, lo, mid + 1)
        return (lo, hi)

    lo0 = jnp.zeros((1, Ns), jnp.int32)
    hi0 = jnp.full((1, Ns), 0x7F800000, jnp.int32)
    _, thr = jax.lax.fori_loop(0, 31, bisect_body, (lo0, hi0))
    c_norm = jnp.sum(jnp.where(bits <= thr, adots, 0.0))

    LxC, LyC, LzC = L3[:, 0:1], L3[:, 1:2], L3[:, 2:3]
    lsqC = LxC * LxC + LyC * LyC + LzC * LzC
    q = (lsqC + ksqR) - (2.0 * LxC) * KxR
    q = q - (2.0 * LyC) * KyR
    q = q - (2.0 * LzC) * KzR
    cnl = LxC * NxR + LyC * NyR + LzC * NzR
    knR = KxR * NxR + KyR * NyR + KzR * NzR
    msum = jnp.zeros((1, Ns), jnp.float32)
    msqsum = jnp.zeros((1, Ns), jnp.float32)
    for k in range(30):
        t = k / 30.0
        a_t = (2.0 * t) * knR + (t * t) * nsqR
        m = jnp.min(q - (2.0 * t) * cnl, axis=0, keepdims=True) + a_t
        msum = msum + m
        msqsum = msqsum + m * m
    c_sw_sum = jnp.sum(msum)
    c_sw_sq = jnp.sum(msqsum)
    n_sw = jnp.float32(30 * Ns)
    sw_mean = c_sw_sum / n_sw
    c_sw_var = c_sw_sq / n_sw - sw_mean * sw_mean

    nn = jnp.sqrt(nsqR + 1e-12)
    c_ndist = jnp.sum(jnp.exp(-(nn - 0.3)) + jnp.where(nn < 0.25, 5.0, 0.0))

    d4 = ((Kx - KxR) ** 2 + (Ky - KyR) ** 2 + (Kz - KzR) ** 2)
    dnx = Nx - NxR
    dny = Ny - NyR
    dnz = Nz - NzR
    smat = jnp.sqrt(dnx * dnx + dny * dny + dnz * dnz + 1e-12)
    iota4 = jax.lax.broadcasted_iota(jnp.int32, (Ns, Ns), 1)

    wex = d4
    acc4 = jnp.zeros((Ns, 1), jnp.float32)
    for _ in range(3):
        m = jnp.min(wex, axis=1, keepdims=True)
        first = jnp.min(jnp.where(wex == m, iota4, Ns), axis=1, keepdims=True)
        sel = iota4 == first
        acc4 = acc4 + jnp.sum(jnp.where(sel, smat, 0.0), axis=1, keepdims=True)
        wex = jnp.where(sel, _BIG, wex)
    c_smooth = jnp.sum(acc4)

    c_rad = jnp.sum(RT)

    for i, v in enumerate((c_s1, c_s2, c_p2s1, c_p2s2, c_rad, c_norm,
                           c_sw_sum, c_sw_var, c_ndist, c_smooth)):
        out_ref[0:1, 0:1, i:i + 1] = jnp.reshape(v, (1, 1, 1))
    out_ref[0:1, 0:1, 10:16] = jnp.zeros((1, 1, 6), jnp.float32)


def kernel(skel_xyz, skel_radius, shape_cmb_features, skel_nori, weights,
           l3_xyz, l3_normals, shape_xyz, A, w0, w1, w2, w3, w4, w5, w6):
    B, Ns, _ = skel_xyz.shape
    Np = shape_xyz.shape[1]
    Nl = l3_xyz.shape[1]

    kt = jnp.transpose(skel_xyz, (0, 2, 1))
    nt = jnp.transpose(skel_nori, (0, 2, 1))
    rt = jnp.transpose(skel_radius, (0, 2, 1))

    out = pl.pallas_call(
        _loss_kernel,
        grid=(B,),
        in_specs=[
            pl.BlockSpec((1, Ns, 3), lambda b: (b, 0, 0)),
            pl.BlockSpec((1, 1, Ns), lambda b: (b, 0, 0)),
            pl.BlockSpec((1, Ns, 3), lambda b: (b, 0, 0)),
            pl.BlockSpec((1, Np, 6), lambda b: (b, 0, 0)),
            pl.BlockSpec((1, Nl, 3), lambda b: (b, 0, 0)),
            pl.BlockSpec((1, 3, Ns), lambda b: (b, 0, 0)),
            pl.BlockSpec((1, 3, Ns), lambda b: (b, 0, 0)),
        ],
        out_specs=pl.BlockSpec((1, 1, 16), lambda b: (b, 0, 0)),
        out_shape=jax.ShapeDtypeStruct((B, 1, 16), jnp.float32),
        compiler_params=pltpu.CompilerParams(
            dimension_semantics=("parallel",)),
    )(skel_xyz, rt, skel_nori, shape_xyz, l3_xyz, kt, nt)

    o = jnp.sum(out, axis=(0, 1))
    loss_sample = o[0] / (Ns * 8.0) + o[1] / float(Np)
    loss_point2sphere = o[2] / float(Np) + o[3] / float(Ns)
    loss_radius = -o[4] / float(Ns)
    loss_normal = o[5] / 30.0 / B
    loss_skelenormal = 50.0 * o[6] / (B * Ns * 30.0) + 500.0 * o[7] / B
    loss_normaldist = o[8] / float(B * Ns)
    loss_normalsmooth = o[9] / (B * Ns * 3.0)
    return (w0 * loss_sample + loss_point2sphere * w1 + loss_radius * w2
            + loss_normal * w4 + loss_skelenormal * w5
            + w6 * loss_normaldist + 0.1 * loss_normalsmooth)
